# baseline (speedup 1.0000x reference)
import functools
import numpy as np
import jax
import jax.numpy as jnp
from jax import lax
from jax.experimental import pallas as pl
from jax.experimental.pallas import tpu as pltpu
from jax.experimental.pallas import tpu_sc as plsc

LAMBDA_P = 0.1

_NC = 2
_NS = 16
_NW = _NC * _NS
_L = 16
_HB = 128


def _sc_histograms(nper):
    nvec = nper // _L
    mesh = plsc.VectorSubcoreMesh(
        core_axis_name="c", subcore_axis_name="s", num_cores=_NC, num_subcores=_NS
    )

    @functools.partial(
        pl.kernel,
        out_type=(
            jax.ShapeDtypeStruct((_NW, _HB), jnp.float32),
            jax.ShapeDtypeStruct((_NW, _HB), jnp.float32),
        ),
        mesh=mesh,
        compiler_params=pltpu.CompilerParams(needs_layout_passes=False),
        scratch_types=[
            pltpu.VMEM((nper,), jnp.int32),
            pltpu.VMEM((_L * _HB,), jnp.float32),
            pltpu.VMEM((_HB,), jnp.float32),
        ],
    )
    def hist_kernel(a_hbm, b_hbm, out_a, out_b, chunk_v, hist_v, row_v):
        wid = lax.axis_index("s") * _NC + lax.axis_index("c")
        base = wid * nper
        lanes = lax.iota(jnp.int32, _L)
        ones = jnp.ones((_L,), jnp.float32)
        zeros = jnp.zeros((_L,), jnp.float32)

        lane_off = lanes * _HB

        def one_array(src_hbm, dst_hbm):
            for r in range(_L * _HB // _L):
                hist_v[pl.ds(r * _L, _L)] = zeros
            pltpu.sync_copy(src_hbm.at[pl.ds(base, nper)], chunk_v)

            def step(i, carry):
                v = chunk_v[pl.ds(i * _L, _L)]
                plsc.addupdate_scatter(hist_v, [lane_off + v], ones)
                return carry

            lax.fori_loop(0, nvec, step, 0)

            for c in range(_HB // _L):
                acc = hist_v[pl.ds(c * _L, _L)]
                for r in range(1, _L):
                    acc = acc + hist_v[pl.ds(r * _HB + c * _L, _L)]
                row_v[pl.ds(c * _L, _L)] = acc
            pltpu.sync_copy(row_v, dst_hbm.at[wid])

        one_array(a_hbm, out_a)
        one_array(b_hbm, out_b)

    return hist_kernel


def _tc_weighted_sum(body_x_ref, pg_ref, o_ref, *, nblocks, const, inv_b):
    i = pl.program_id(0)
    ps = jnp.sum(body_x_ref[...] * pg_ref[...])

    @pl.when(i == 0)
    def _():
        o_ref[0, 0] = jnp.float32(0.0)

    acc = o_ref[0, 0] + ps

    @pl.when(i < nblocks - 1)
    def _():
        o_ref[0, 0] = acc

    @pl.when(i == nblocks - 1)
    def _():
        o_ref[0, 0] = jnp.float32(const) - acc * jnp.float32(inv_b)


def _tc_combine(pgl_ref, hp_ref, he_ref, o_ref, *, steps):
    hp = jnp.sum(hp_ref[...], axis=0, keepdims=True)
    he = jnp.sum(he_ref[...], axis=0, keepdims=True)
    cols = lax.broadcasted_iota(jnp.int32, (1, _HB), 1)
    valid = (cols >= 1) & (cols <= steps)
    pred = jnp.where(valid, hp, 0.0)
    emp = jnp.where(valid, he, 0.0)
    pred_n = pred / jnp.maximum(jnp.sum(pred), 1e-12)
    emp_n = emp / jnp.maximum(jnp.sum(emp), 1e-12)
    safe = jnp.where(emp_n > 0, emp_n, 1.0)
    xlogy = jnp.where(emp_n > 0, emp_n * jnp.log(safe), 0.0)
    eloss = jnp.sum(xlogy - emp_n * pred_n) / jnp.float32(steps)
    o_ref[0, 0] = pgl_ref[0, 0] + eloss


def kernel(p_halts, halt_steps, responses, response_times):
    del responses
    batch, steps = p_halts.shape

    blk = batch
    for cand in (10000, 8000, 5000, 4000, 2000, 1000, 500):
        if batch % cand == 0:
            blk = cand
            break
    nblocks = batch // blk

    pg = (LAMBDA_P * np.power(1.0 - LAMBDA_P, np.arange(steps))).astype(np.float32)
    const = float(np.sum(pg * np.log(pg)))
    pg2d = jnp.asarray(pg.reshape(1, steps))

    pgl = pl.pallas_call(
        functools.partial(
            _tc_weighted_sum, nblocks=nblocks, const=const, inv_b=1.0 / batch
        ),
        grid=(nblocks,),
        in_specs=[
            pl.BlockSpec((blk, steps), lambda i: (i, 0)),
            pl.BlockSpec((1, steps), lambda i: (0, 0)),
        ],
        out_specs=pl.BlockSpec(memory_space=pltpu.MemorySpace.SMEM),
        out_shape=jax.ShapeDtypeStruct((1, 1), jnp.float32),
        compiler_params=pltpu.CompilerParams(
            dimension_semantics=("arbitrary",),
        ),
    )(p_halts, pg2d)

    n = halt_steps.shape[0]
    nper = (-(-n // _NW) + _L - 1) // _L * _L
    npad = nper * _NW
    fill = jnp.full((npad - n,), _HB - 1, jnp.int32)
    hs_p = jnp.concatenate([halt_steps, fill])
    rt_p = jnp.concatenate([response_times, fill])
    hp_parts, he_parts = _sc_histograms(nper)(hs_p, rt_p)

    out = pl.pallas_call(
        functools.partial(_tc_combine, steps=steps),
        in_specs=[
            pl.BlockSpec(memory_space=pltpu.MemorySpace.SMEM),
            pl.BlockSpec((_NW, _HB), lambda: (0, 0)),
            pl.BlockSpec((_NW, _HB), lambda: (0, 0)),
        ],
        out_specs=pl.BlockSpec(memory_space=pltpu.MemorySpace.SMEM),
        out_shape=jax.ShapeDtypeStruct((1, 1), jnp.float32),
    )(pgl, hp_parts, he_parts)
    return out[0, 0]


# speedup vs baseline: 3.6030x; 3.6030x over previous
import functools
import numpy as np
import jax
import jax.numpy as jnp
from jax import lax
from jax.experimental import pallas as pl
from jax.experimental.pallas import tpu as pltpu
from jax.experimental.pallas import tpu_sc as plsc

LAMBDA_P = 0.1

_NC = 2
_NS = 16
_NW = _NC * _NS
_L = 16
_HB = 128


def _sc_histograms(nper):
    nvec = nper // _L
    mesh = plsc.VectorSubcoreMesh(
        core_axis_name="c", subcore_axis_name="s", num_cores=_NC, num_subcores=_NS
    )

    @functools.partial(
        pl.kernel,
        out_type=(
            jax.ShapeDtypeStruct((_NW, _HB), jnp.float32),
            jax.ShapeDtypeStruct((_NW, _HB), jnp.float32),
        ),
        mesh=mesh,
        compiler_params=pltpu.CompilerParams(needs_layout_passes=False),
        scratch_types=[
            pltpu.VMEM((nper,), jnp.int32),
            pltpu.VMEM((_L * _HB,), jnp.float32),
            pltpu.VMEM((_HB,), jnp.float32),
        ],
    )
    def hist_kernel(a_hbm, b_hbm, out_a, out_b, chunk_v, hist_v, row_v):
        wid = lax.axis_index("s") * _NC + lax.axis_index("c")
        base = wid * nper
        lanes = lax.iota(jnp.int32, _L)
        ones = jnp.ones((_L,), jnp.float32)
        zeros = jnp.zeros((_L,), jnp.float32)

        lane_off = lanes * _HB

        def one_array(src_hbm, dst_hbm):
            for r in range(_L * _HB // _L):
                hist_v[pl.ds(r * _L, _L)] = zeros
            pltpu.sync_copy(src_hbm.at[pl.ds(base, nper)], chunk_v)

            def step(i, carry):
                v = chunk_v[pl.ds(i * _L, _L)]
                plsc.addupdate_scatter(hist_v, [lane_off + v], ones)
                return carry

            lax.fori_loop(0, nvec, step, 0)

            for c in range(_HB // _L):
                acc = hist_v[pl.ds(c * _L, _L)]
                for r in range(1, _L):
                    acc = acc + hist_v[pl.ds(r * _HB + c * _L, _L)]
                row_v[pl.ds(c * _L, _L)] = acc
            pltpu.sync_copy(row_v, dst_hbm.at[wid])

        one_array(a_hbm, out_a)
        one_array(b_hbm, out_b)

    return hist_kernel


def _tc_weighted_sum(x_ref, pg_ref, o_ref, *, nblocks, blk, nvalid, const, inv_b):
    i = pl.program_id(0)
    row = jax.lax.dot_general(
        pg_ref[...], x_ref[...], (((1,), (0,)), ((), ())),
        preferred_element_type=jnp.float32,
    )
    ids = i * blk + lax.broadcasted_iota(jnp.int32, (1, blk), 1)
    ps = jnp.sum(jnp.where(ids < nvalid, row, 0.0))

    @pl.when(i == 0)
    def _():
        o_ref[0, 0] = jnp.float32(0.0)

    acc = o_ref[0, 0] + ps

    @pl.when(i < nblocks - 1)
    def _():
        o_ref[0, 0] = acc

    @pl.when(i == nblocks - 1)
    def _():
        o_ref[0, 0] = jnp.float32(const) - acc * jnp.float32(inv_b)


def _tc_combine(pgl_ref, hp_ref, he_ref, o_ref, *, steps):
    hp = jnp.sum(hp_ref[...], axis=0, keepdims=True)
    he = jnp.sum(he_ref[...], axis=0, keepdims=True)
    cols = lax.broadcasted_iota(jnp.int32, (1, _HB), 1)
    valid = (cols >= 1) & (cols <= steps)
    pred = jnp.where(valid, hp, 0.0)
    emp = jnp.where(valid, he, 0.0)
    pred_n = pred / jnp.maximum(jnp.sum(pred), 1e-12)
    emp_n = emp / jnp.maximum(jnp.sum(emp), 1e-12)
    safe = jnp.where(emp_n > 0, emp_n, 1.0)
    xlogy = jnp.where(emp_n > 0, emp_n * jnp.log(safe), 0.0)
    eloss = jnp.sum(xlogy - emp_n * pred_n) / jnp.float32(steps)
    o_ref[0, 0] = pgl_ref[0, 0] + eloss


def kernel(p_halts, halt_steps, responses, response_times):
    del responses
    batch, steps = p_halts.shape

    blk = 12800
    nblocks = -(-batch // blk)

    pg = (LAMBDA_P * np.power(1.0 - LAMBDA_P, np.arange(steps))).astype(np.float32)
    const = float(np.sum(pg * np.log(pg)))
    pg2d = jnp.asarray(pg.reshape(1, steps))

    pgl = pl.pallas_call(
        functools.partial(
            _tc_weighted_sum, nblocks=nblocks, blk=blk, nvalid=batch,
            const=const, inv_b=1.0 / batch,
        ),
        grid=(nblocks,),
        in_specs=[
            pl.BlockSpec((steps, blk), lambda i: (0, i)),
            pl.BlockSpec((1, steps), lambda i: (0, 0)),
        ],
        out_specs=pl.BlockSpec(memory_space=pltpu.MemorySpace.SMEM),
        out_shape=jax.ShapeDtypeStruct((1, 1), jnp.float32),
        compiler_params=pltpu.CompilerParams(
            dimension_semantics=("arbitrary",),
        ),
    )(p_halts.T, pg2d)

    n = halt_steps.shape[0]
    nper = (-(-n // _NW) + _L - 1) // _L * _L
    npad = nper * _NW
    fill = jnp.full((npad - n,), _HB - 1, jnp.int32)
    hs_p = jnp.concatenate([halt_steps, fill])
    rt_p = jnp.concatenate([response_times, fill])
    hp_parts, he_parts = _sc_histograms(nper)(hs_p, rt_p)

    out = pl.pallas_call(
        functools.partial(_tc_combine, steps=steps),
        in_specs=[
            pl.BlockSpec(memory_space=pltpu.MemorySpace.SMEM),
            pl.BlockSpec((_NW, _HB), lambda: (0, 0)),
            pl.BlockSpec((_NW, _HB), lambda: (0, 0)),
        ],
        out_specs=pl.BlockSpec(memory_space=pltpu.MemorySpace.SMEM),
        out_shape=jax.ShapeDtypeStruct((1, 1), jnp.float32),
    )(pgl, hp_parts, he_parts)
    return out[0, 0]


# speedup vs baseline: 3.7485x; 1.0404x over previous
import functools
import numpy as np
import jax
import jax.numpy as jnp
from jax import lax
from jax.experimental import pallas as pl
from jax.experimental.pallas import tpu as pltpu
from jax.experimental.pallas import tpu_sc as plsc

LAMBDA_P = 0.1

_NC = 2
_NS = 16
_NW = _NC * _NS
_L = 16
_HB = 128


def _sc_histograms(nper, ntail):
    nvec = nper // _L
    nvec_tail = ntail // _L
    mesh = plsc.VectorSubcoreMesh(
        core_axis_name="c", subcore_axis_name="s", num_cores=_NC, num_subcores=_NS
    )

    @functools.partial(
        pl.kernel,
        out_type=(
            jax.ShapeDtypeStruct((_NW, _HB), jnp.float32),
            jax.ShapeDtypeStruct((_NW, _HB), jnp.float32),
        ),
        mesh=mesh,
        compiler_params=pltpu.CompilerParams(needs_layout_passes=False),
        scratch_types=[
            pltpu.VMEM((nper,), jnp.int32),
            pltpu.VMEM((_L * _HB,), jnp.float32),
            pltpu.VMEM((_HB,), jnp.float32),
        ],
    )
    def hist_kernel(a_hbm, b_hbm, out_a, out_b, chunk_v, hist_v, row_v):
        wid = lax.axis_index("s") * _NC + lax.axis_index("c")
        base = wid * nper
        lanes = lax.iota(jnp.int32, _L)
        ones = jnp.ones((_L,), jnp.float32)
        zeros = jnp.zeros((_L,), jnp.float32)

        lane_off = lanes * _HB
        is_tail = wid == _NW - 1
        my_nvec = jnp.where(is_tail, nvec_tail, nvec)

        def one_array(src_hbm, dst_hbm):
            for r in range(_L * _HB // _L):
                hist_v[pl.ds(r * _L, _L)] = zeros

            @pl.when(jnp.logical_not(is_tail))
            def _():
                pltpu.sync_copy(src_hbm.at[pl.ds(base, nper)], chunk_v)

            @pl.when(is_tail)
            def _():
                pltpu.sync_copy(
                    src_hbm.at[pl.ds(base, ntail)], chunk_v.at[pl.ds(0, ntail)]
                )

            def step(i, carry):
                v = chunk_v[pl.ds(i * _L, _L)]
                plsc.addupdate_scatter(hist_v, [lane_off + v], ones)
                return carry

            lax.fori_loop(0, my_nvec, step, 0)

            for c in range(_HB // _L):
                acc = hist_v[pl.ds(c * _L, _L)]
                for r in range(1, _L):
                    acc = acc + hist_v[pl.ds(r * _HB + c * _L, _L)]
                row_v[pl.ds(c * _L, _L)] = acc
            pltpu.sync_copy(row_v, dst_hbm.at[wid])

        one_array(a_hbm, out_a)
        one_array(b_hbm, out_b)

    return hist_kernel


def _tc_weighted_sum(x_ref, pg_ref, o_ref, *, nblocks, blk, nvalid, const, inv_b):
    i = pl.program_id(0)
    row = jax.lax.dot_general(
        pg_ref[...], x_ref[...], (((1,), (0,)), ((), ())),
        preferred_element_type=jnp.float32,
    )
    ids = i * blk + lax.broadcasted_iota(jnp.int32, (1, blk), 1)
    ps = jnp.sum(jnp.where(ids < nvalid, row, 0.0))

    @pl.when(i == 0)
    def _():
        o_ref[0, 0] = jnp.float32(0.0)

    acc = o_ref[0, 0] + ps

    @pl.when(i < nblocks - 1)
    def _():
        o_ref[0, 0] = acc

    @pl.when(i == nblocks - 1)
    def _():
        o_ref[0, 0] = jnp.float32(const) - acc * jnp.float32(inv_b)


def _tc_combine(pgl_ref, hp_ref, he_ref, o_ref, *, steps):
    hp = jnp.sum(hp_ref[...], axis=0, keepdims=True)
    he = jnp.sum(he_ref[...], axis=0, keepdims=True)
    cols = lax.broadcasted_iota(jnp.int32, (1, _HB), 1)
    valid = (cols >= 1) & (cols <= steps)
    pred = jnp.where(valid, hp, 0.0)
    emp = jnp.where(valid, he, 0.0)
    pred_n = pred / jnp.maximum(jnp.sum(pred), 1e-12)
    emp_n = emp / jnp.maximum(jnp.sum(emp), 1e-12)
    safe = jnp.where(emp_n > 0, emp_n, 1.0)
    xlogy = jnp.where(emp_n > 0, emp_n * jnp.log(safe), 0.0)
    eloss = jnp.sum(xlogy - emp_n * pred_n) / jnp.float32(steps)
    o_ref[0, 0] = pgl_ref[0, 0] + eloss


def kernel(p_halts, halt_steps, responses, response_times):
    del responses
    batch, steps = p_halts.shape

    blk = 12800
    nblocks = -(-batch // blk)

    pg = (LAMBDA_P * np.power(1.0 - LAMBDA_P, np.arange(steps))).astype(np.float32)
    const = float(np.sum(pg * np.log(pg)))
    pg2d = jnp.asarray(pg.reshape(1, steps))

    pgl = pl.pallas_call(
        functools.partial(
            _tc_weighted_sum, nblocks=nblocks, blk=blk, nvalid=batch,
            const=const, inv_b=1.0 / batch,
        ),
        grid=(nblocks,),
        in_specs=[
            pl.BlockSpec((steps, blk), lambda i: (0, i)),
            pl.BlockSpec((1, steps), lambda i: (0, 0)),
        ],
        out_specs=pl.BlockSpec(memory_space=pltpu.MemorySpace.SMEM),
        out_shape=jax.ShapeDtypeStruct((1, 1), jnp.float32),
        compiler_params=pltpu.CompilerParams(
            dimension_semantics=("arbitrary",),
        ),
    )(p_halts.T, pg2d)

    n = halt_steps.shape[0]
    nper = (-(-n // _NW) + _L - 1) // _L * _L
    ntail = n - (_NW - 1) * nper
    assert 0 < ntail <= nper and ntail % _L == 0 and nper % 8 == 0
    hp_parts, he_parts = _sc_histograms(nper, ntail)(halt_steps, response_times)

    out = pl.pallas_call(
        functools.partial(_tc_combine, steps=steps),
        in_specs=[
            pl.BlockSpec(memory_space=pltpu.MemorySpace.SMEM),
            pl.BlockSpec((_NW, _HB), lambda: (0, 0)),
            pl.BlockSpec((_NW, _HB), lambda: (0, 0)),
        ],
        out_specs=pl.BlockSpec(memory_space=pltpu.MemorySpace.SMEM),
        out_shape=jax.ShapeDtypeStruct((1, 1), jnp.float32),
    )(pgl, hp_parts, he_parts)
    return out[0, 0]


# speedup vs baseline: 3.9816x; 1.0622x over previous
import functools
import numpy as np
import jax
import jax.numpy as jnp
from jax import lax
from jax.experimental import pallas as pl
from jax.experimental.pallas import tpu as pltpu
from jax.experimental.pallas import tpu_sc as plsc

LAMBDA_P = 0.1

_NC = 2
_NS = 16
_NW = _NC * _NS
_L = 16
_HB = 128


def _sc_histograms(nper, ntail):
    nvec = nper // _L
    nvec_tail = ntail // _L
    mesh = plsc.VectorSubcoreMesh(
        core_axis_name="c", subcore_axis_name="s", num_cores=_NC, num_subcores=_NS
    )

    @functools.partial(
        pl.kernel,
        out_type=(
            jax.ShapeDtypeStruct((_NW, _HB), jnp.float32),
            jax.ShapeDtypeStruct((_NW, _HB), jnp.float32),
        ),
        mesh=mesh,
        compiler_params=pltpu.CompilerParams(needs_layout_passes=False),
        scratch_types=[
            pltpu.VMEM((nper,), jnp.int32),
            pltpu.VMEM((_L * _HB,), jnp.float32),
            pltpu.VMEM((_HB,), jnp.float32),
        ],
    )
    def hist_kernel(a_hbm, b_hbm, out_a, out_b, chunk_v, hist_v, row_v):
        wid = lax.axis_index("s") * _NC + lax.axis_index("c")
        base = wid * nper
        lanes = lax.iota(jnp.int32, _L)
        ones = jnp.ones((_L,), jnp.float32)
        zeros = jnp.zeros((_L,), jnp.float32)

        lane_off = lanes * _HB
        is_tail = wid == _NW - 1
        my_nvec = jnp.where(is_tail, nvec_tail, nvec)

        def one_array(src_hbm, dst_hbm):
            for r in range(_L * _HB // _L):
                hist_v[pl.ds(r * _L, _L)] = zeros

            @pl.when(jnp.logical_not(is_tail))
            def _():
                pltpu.sync_copy(src_hbm.at[pl.ds(base, nper)], chunk_v)

            @pl.when(is_tail)
            def _():
                pltpu.sync_copy(
                    src_hbm.at[pl.ds(base, ntail)], chunk_v.at[pl.ds(0, ntail)]
                )

            def step(i, carry):
                v = chunk_v[pl.ds(i * _L, _L)]
                plsc.addupdate_scatter(hist_v, [lane_off + v], ones)
                return carry

            lax.fori_loop(0, my_nvec, step, 0)

            for c in range(_HB // _L):
                acc = hist_v[pl.ds(c * _L, _L)]
                for r in range(1, _L):
                    acc = acc + hist_v[pl.ds(r * _HB + c * _L, _L)]
                row_v[pl.ds(c * _L, _L)] = acc
            pltpu.sync_copy(row_v, dst_hbm.at[wid])

        one_array(a_hbm, out_a)
        one_array(b_hbm, out_b)

    return hist_kernel


def _tc_weighted_sum(x_ref, pg_ref, o_ref, *, nblocks, blk, nvalid, const, inv_b):
    i = pl.program_id(0)
    row = jax.lax.dot_general(
        pg_ref[...], x_ref[...], (((1,), (0,)), ((), ())),
        preferred_element_type=jnp.float32,
    )
    ids = i * blk + lax.broadcasted_iota(jnp.int32, (1, blk), 1)
    ps = jnp.sum(jnp.where(ids < nvalid, row, 0.0))

    @pl.when(i == 0)
    def _():
        o_ref[0, 0] = jnp.float32(0.0)

    acc = o_ref[0, 0] + ps

    @pl.when(i < nblocks - 1)
    def _():
        o_ref[0, 0] = acc

    @pl.when(i == nblocks - 1)
    def _():
        o_ref[0, 0] = jnp.float32(const) - acc * jnp.float32(inv_b)


def _tc_combine(pgl_ref, hp_ref, he_ref, o_ref, *, steps):
    hp = jnp.sum(hp_ref[...], axis=0, keepdims=True)
    he = jnp.sum(he_ref[...], axis=0, keepdims=True)
    cols = lax.broadcasted_iota(jnp.int32, (1, _HB), 1)
    valid = (cols >= 1) & (cols <= steps)
    pred = jnp.where(valid, hp, 0.0)
    emp = jnp.where(valid, he, 0.0)
    pred_n = pred / jnp.maximum(jnp.sum(pred), 1e-12)
    emp_n = emp / jnp.maximum(jnp.sum(emp), 1e-12)
    safe = jnp.where(emp_n > 0, emp_n, 1.0)
    xlogy = jnp.where(emp_n > 0, emp_n * jnp.log(safe), 0.0)
    eloss = jnp.sum(xlogy - emp_n * pred_n) / jnp.float32(steps)
    o_ref[0, 0] = pgl_ref[0, 0] + eloss


def kernel(p_halts, halt_steps, responses, response_times):
    del responses
    batch, steps = p_halts.shape

    blk = 25600
    nblocks = -(-batch // blk)

    pg = (LAMBDA_P * np.power(1.0 - LAMBDA_P, np.arange(steps))).astype(np.float32)
    const = float(np.sum(pg * np.log(pg)))
    pg2d = jnp.asarray(pg.reshape(1, steps))

    pgl = pl.pallas_call(
        functools.partial(
            _tc_weighted_sum, nblocks=nblocks, blk=blk, nvalid=batch,
            const=const, inv_b=1.0 / batch,
        ),
        grid=(nblocks,),
        in_specs=[
            pl.BlockSpec((steps, blk), lambda i: (0, i)),
            pl.BlockSpec((1, steps), lambda i: (0, 0)),
        ],
        out_specs=pl.BlockSpec(memory_space=pltpu.MemorySpace.SMEM),
        out_shape=jax.ShapeDtypeStruct((1, 1), jnp.float32),
        compiler_params=pltpu.CompilerParams(
            dimension_semantics=("arbitrary",),
        ),
    )(p_halts.T, pg2d)

    n = halt_steps.shape[0]
    nper = (-(-n // _NW) + _L - 1) // _L * _L
    ntail = n - (_NW - 1) * nper
    assert 0 < ntail <= nper and ntail % _L == 0 and nper % 8 == 0
    hp_parts, he_parts = _sc_histograms(nper, ntail)(halt_steps, response_times)

    out = pl.pallas_call(
        functools.partial(_tc_combine, steps=steps),
        in_specs=[
            pl.BlockSpec(memory_space=pltpu.MemorySpace.SMEM),
            pl.BlockSpec((_NW, _HB), lambda: (0, 0)),
            pl.BlockSpec((_NW, _HB), lambda: (0, 0)),
        ],
        out_specs=pl.BlockSpec(memory_space=pltpu.MemorySpace.SMEM),
        out_shape=jax.ShapeDtypeStruct((1, 1), jnp.float32),
    )(pgl, hp_parts, he_parts)
    return out[0, 0]
